# baseline (speedup 1.0000x reference)
import functools

import jax
import jax.numpy as jnp
from jax import lax
from jax.experimental import pallas as pl
from jax.experimental.pallas import tpu as pltpu
from jax.experimental.pallas import tpu_sc as plsc

N = 10000
E = 320000
D = 128

NC = 2
NS = 16
NW = NC * NS

NPAD = 10240
ROWS_PER_TILE = NPAD // NS
CHUNK = 128
NSEG = 4
CPS = 40
EW_T = NSEG * CPS * CHUNK
E_PAD = NS * EW_T
EWD = E_PAD // NW

_mesh = plsc.VectorSubcoreMesh(core_axis_name="c", subcore_axis_name="s")


@functools.partial(
    pl.kernel,
    out_type=jax.ShapeDtypeStruct((NW, NPAD), jnp.float32),
    mesh=_mesh,
    scratch_types=[
        pltpu.VMEM((EWD,), jnp.int32),
        pltpu.VMEM((NPAD,), jnp.float32),
    ],
    compiler_params=pltpu.CompilerParams(needs_layout_passes=False),
)
def _deg_kernel(dst_hbm, deg_out, idx_v, acc_v):
    c = lax.axis_index("c")
    s = lax.axis_index("s")
    w = c * NS + s

    def zero_body(i, _):
        acc_v[pl.ds(i * 16, 16)] = jnp.zeros((16,), jnp.float32)
        return 0

    lax.fori_loop(0, NPAD // 16, zero_body, 0)
    pltpu.sync_copy(dst_hbm.at[c, s], idx_v)

    ones = jnp.ones((16,), jnp.float32)

    def body(j, _):
        idx16 = idx_v[pl.ds(j * 16, 16)]
        plsc.addupdate_scatter(acc_v, [idx16], ones)
        return 0

    lax.fori_loop(0, EWD // 16, body, 0)
    pltpu.sync_copy(acc_v, deg_out.at[w])


@functools.partial(
    pl.kernel,
    out_type=jax.ShapeDtypeStruct((NPAD, D), jnp.float32),
    mesh=_mesh,
    scratch_types=[
        pltpu.VMEM_SHARED((NPAD, D), jnp.float32),
        pltpu.VMEM((CPS, CHUNK), jnp.int32),
        pltpu.VMEM((CPS, CHUNK), jnp.int32),
        pltpu.VMEM((CHUNK, D), jnp.float32),
        pltpu.VMEM((CHUNK, D), jnp.float32),
        pltpu.SemaphoreType.DMA((2,)),
    ],
    compiler_params=pltpu.CompilerParams(needs_layout_passes=False),
)
def _hop_kernel(x_hbm, src_hbm, dst_hbm, zeros_hbm,
                part_out, acc_sh, srcix, dstix, rows0, rows1, sem):
    c = lax.axis_index("c")
    s = lax.axis_index("s")
    row0 = s * ROWS_PER_TILE
    bufs = (rows0, rows1)

    @pl.when(c == 0)
    def _():
        pltpu.sync_copy(zeros_hbm, acc_sh.at[pl.ds(row0, ROWS_PER_TILE)])

    plsc.subcore_barrier()

    @pl.when(c == 0)
    def _():
        for seg in range(NSEG):
            pltpu.sync_copy(src_hbm.at[s, seg], srcix)
            pltpu.sync_copy(dst_hbm.at[s, seg], dstix)

            pltpu.async_copy(x_hbm.at[srcix.at[0]], bufs[0], sem.at[0])
            pltpu.async_copy(x_hbm.at[srcix.at[1]], bufs[1], sem.at[1])

            def body(p, _):
                g = p * 2
                for b in range(2):
                    j = g + b
                    pltpu.make_async_copy(
                        x_hbm.at[srcix.at[j]], bufs[b], sem.at[b]).wait()
                    pltpu.sync_copy(
                        bufs[b], acc_sh.at[dstix.at[j]], add=True)
                    pltpu.async_copy(
                        x_hbm.at[srcix.at[j + 2]], bufs[b], sem.at[b])
                return 0

            lax.fori_loop(0, (CPS - 2) // 2, body, 0)
            for b in range(2):
                j = CPS - 2 + b
                pltpu.make_async_copy(
                    x_hbm.at[srcix.at[j]], bufs[b], sem.at[b]).wait()
                pltpu.sync_copy(bufs[b], acc_sh.at[dstix.at[j]], add=True)

    plsc.subcore_barrier()

    @pl.when(c == 0)
    def _():
        pltpu.sync_copy(acc_sh.at[pl.ds(row0, ROWS_PER_TILE)],
                        part_out.at[pl.ds(row0, ROWS_PER_TILE)])


_BLK = 512
_GRID = NPAD // _BLK


def _norm_x0_body(dp_ref, feat_ref, x0_ref, norm_ref, norm2_ref):
    deg = jnp.sum(dp_ref[...], axis=0)
    nrm = lax.rsqrt(jnp.clip(deg, 1.0, None))
    norm_ref[...] = nrm
    norm2_ref[...] = nrm * nrm
    x0_ref[...] = feat_ref[...] * nrm[:, None]


def _scale_sum_body(part_ref, sc_ref, out_ref):
    out_ref[...] = part_ref[...] * sc_ref[...][:, None]


def _final_body(part_ref, norm_ref, feat_ref, wfc_ref, wres_ref,
                bfc_ref, bres_ref, out_ref):
    h = part_ref[...] * norm_ref[...][:, None]
    out_ref[...] = (
        jnp.dot(h, wfc_ref[...], preferred_element_type=jnp.float32)
        + jnp.dot(feat_ref[...], wres_ref[...],
                  preferred_element_type=jnp.float32)
        + bfc_ref[...][None, :] + bres_ref[...][None, :]
    )


_norm_x0 = pl.pallas_call(
    _norm_x0_body,
    grid=(_GRID,),
    in_specs=[
        pl.BlockSpec((NW, _BLK), lambda i: (0, i)),
        pl.BlockSpec((_BLK, D), lambda i: (i, 0)),
    ],
    out_specs=[
        pl.BlockSpec((_BLK, D), lambda i: (i, 0)),
        pl.BlockSpec((_BLK,), lambda i: (i,)),
        pl.BlockSpec((_BLK,), lambda i: (i,)),
    ],
    out_shape=[
        jax.ShapeDtypeStruct((NPAD, D), jnp.float32),
        jax.ShapeDtypeStruct((NPAD,), jnp.float32),
        jax.ShapeDtypeStruct((NPAD,), jnp.float32),
    ],
)

_scale_sum = pl.pallas_call(
    _scale_sum_body,
    grid=(_GRID,),
    in_specs=[
        pl.BlockSpec((_BLK, D), lambda i: (i, 0)),
        pl.BlockSpec((_BLK,), lambda i: (i,)),
    ],
    out_specs=pl.BlockSpec((_BLK, D), lambda i: (i, 0)),
    out_shape=jax.ShapeDtypeStruct((NPAD, D), jnp.float32),
)

_final = pl.pallas_call(
    _final_body,
    grid=(_GRID,),
    in_specs=[
        pl.BlockSpec((_BLK, D), lambda i: (i, 0)),
        pl.BlockSpec((_BLK,), lambda i: (i,)),
        pl.BlockSpec((_BLK, D), lambda i: (i, 0)),
        pl.BlockSpec((D, D), lambda i: (0, 0)),
        pl.BlockSpec((D, D), lambda i: (0, 0)),
        pl.BlockSpec((D,), lambda i: (0,)),
        pl.BlockSpec((D,), lambda i: (0,)),
    ],
    out_specs=pl.BlockSpec((_BLK, D), lambda i: (i, 0)),
    out_shape=jax.ShapeDtypeStruct((NPAD, D), jnp.float32),
)


@jax.jit
def kernel(feat, edge_index, W_fc, b_fc, W_res, b_res):
    src = jnp.concatenate(
        [edge_index[0], jnp.zeros((E_PAD - E,), jnp.int32)])
    pad_dst = N + (jnp.arange(E_PAD - E, dtype=jnp.int32) % (NPAD - N))
    dst = jnp.concatenate([edge_index[1], pad_dst])
    src4 = src.reshape(NS, NSEG, CPS, CHUNK)
    dst4 = dst.reshape(NS, NSEG, CPS, CHUNK)
    dst3 = dst.reshape(NC, NS, EWD)

    feat_pad = jnp.zeros((NPAD, D), jnp.float32).at[:N].set(feat)
    zeros_blk = jnp.zeros((ROWS_PER_TILE, D), jnp.float32)

    deg_parts = _deg_kernel(dst3)
    x0, norm, norm2 = _norm_x0(deg_parts, feat_pad)

    s1 = _hop_kernel(x0, src4, dst4, zeros_blk)
    x1 = _scale_sum(s1, norm2)
    s2 = _hop_kernel(x1, src4, dst4, zeros_blk)

    out = _final(s2, norm, feat_pad, W_fc, W_res, b_fc, b_res)
    return out[:N]


# speedup vs baseline: 1.4254x; 1.4254x over previous
import functools

import jax
import jax.numpy as jnp
from jax import lax
from jax.experimental import pallas as pl
from jax.experimental.pallas import tpu as pltpu
from jax.experimental.pallas import tpu_sc as plsc

N = 10000
E = 320000
D = 128

NC = 2
NS = 16
NW = NC * NS

NPAD = 10240
ROWS_PER_TILE = NPAD // NS
CHUNK = 128
CHUNKS_PER_W = 79
EW = CHUNKS_PER_W * CHUNK
E_PAD = NW * EW

_mesh = plsc.VectorSubcoreMesh(core_axis_name="c", subcore_axis_name="s")


@functools.partial(
    pl.kernel,
    out_type=jax.ShapeDtypeStruct((NW, NPAD), jnp.float32),
    mesh=_mesh,
    scratch_types=[
        pltpu.VMEM((EW,), jnp.int32),
        pltpu.VMEM((NPAD,), jnp.float32),
    ],
    compiler_params=pltpu.CompilerParams(needs_layout_passes=False),
)
def _deg_kernel(dst_hbm, deg_out, idx_v, acc_v):
    c = lax.axis_index("c")
    s = lax.axis_index("s")
    w = c * NS + s

    def zero_body(i, _):
        acc_v[pl.ds(i * 16, 16)] = jnp.zeros((16,), jnp.float32)
        return 0

    lax.fori_loop(0, NPAD // 16, zero_body, 0)
    pltpu.sync_copy(dst_hbm.at[c, s], idx_v)

    ones = jnp.ones((16,), jnp.float32)

    def body(j, _):
        idx16 = idx_v[pl.ds(j * 16, 16)]
        plsc.addupdate_scatter(acc_v, [idx16], ones)
        return 0

    lax.fori_loop(0, EW // 16, body, 0)
    pltpu.sync_copy(acc_v, deg_out.at[w])


@functools.partial(
    pl.kernel,
    out_type=jax.ShapeDtypeStruct((NC, NPAD, D), jnp.float32),
    mesh=_mesh,
    scratch_types=[
        pltpu.VMEM_SHARED((NPAD, D), jnp.float32),
        pltpu.VMEM((CHUNKS_PER_W, CHUNK), jnp.int32),
        pltpu.VMEM((CHUNKS_PER_W, CHUNK), jnp.int32),
        pltpu.VMEM((CHUNK, D), jnp.float32),
        pltpu.SemaphoreType.DMA,
    ],
    compiler_params=pltpu.CompilerParams(needs_layout_passes=False),
)
def _hop_kernel(x_hbm, src_hbm, dst_hbm, zeros_hbm, part_out,
                acc_sh, srcix, dstix, rows, sem):
    c = lax.axis_index("c")
    s = lax.axis_index("s")
    row0 = s * ROWS_PER_TILE

    pltpu.sync_copy(zeros_hbm, acc_sh.at[pl.ds(row0, ROWS_PER_TILE)])
    pltpu.sync_copy(src_hbm.at[c, s], srcix)
    pltpu.sync_copy(dst_hbm.at[c, s], dstix)
    plsc.subcore_barrier()

    def body(j, _):
        pltpu.async_copy(x_hbm.at[srcix.at[j]], rows, sem).wait()
        pltpu.sync_copy(rows, acc_sh.at[dstix.at[j]], add=True)
        return 0

    lax.fori_loop(0, CHUNKS_PER_W, body, 0)
    plsc.subcore_barrier()
    pltpu.sync_copy(acc_sh.at[pl.ds(row0, ROWS_PER_TILE)],
                    part_out.at[c, pl.ds(row0, ROWS_PER_TILE)])


_BLK = 512
_GRID = NPAD // _BLK


def _norm_x0_body(dp_ref, feat_ref, x0_ref, norm_ref, norm2_ref):
    deg = jnp.sum(dp_ref[...], axis=0)
    nrm = lax.rsqrt(jnp.clip(deg, 1.0, None))
    norm_ref[...] = nrm
    norm2_ref[...] = nrm * nrm
    x0_ref[...] = feat_ref[...] * nrm[:, None]


def _scale_sum_body(part_ref, sc_ref, out_ref):
    out_ref[...] = (part_ref[0] + part_ref[1]) * sc_ref[...][:, None]


def _final_body(part_ref, norm_ref, feat_ref, wfc_ref, wres_ref,
                bfc_ref, bres_ref, out_ref):
    h = (part_ref[0] + part_ref[1]) * norm_ref[...][:, None]
    out_ref[...] = (
        jnp.dot(h, wfc_ref[...], preferred_element_type=jnp.float32)
        + jnp.dot(feat_ref[...], wres_ref[...],
                  preferred_element_type=jnp.float32)
        + bfc_ref[...][None, :] + bres_ref[...][None, :]
    )


_norm_x0 = pl.pallas_call(
    _norm_x0_body,
    grid=(_GRID,),
    in_specs=[
        pl.BlockSpec((NW, _BLK), lambda i: (0, i)),
        pl.BlockSpec((_BLK, D), lambda i: (i, 0)),
    ],
    out_specs=[
        pl.BlockSpec((_BLK, D), lambda i: (i, 0)),
        pl.BlockSpec((_BLK,), lambda i: (i,)),
        pl.BlockSpec((_BLK,), lambda i: (i,)),
    ],
    out_shape=[
        jax.ShapeDtypeStruct((NPAD, D), jnp.float32),
        jax.ShapeDtypeStruct((NPAD,), jnp.float32),
        jax.ShapeDtypeStruct((NPAD,), jnp.float32),
    ],
)

_scale_sum = pl.pallas_call(
    _scale_sum_body,
    grid=(_GRID,),
    in_specs=[
        pl.BlockSpec((NC, _BLK, D), lambda i: (0, i, 0)),
        pl.BlockSpec((_BLK,), lambda i: (i,)),
    ],
    out_specs=pl.BlockSpec((_BLK, D), lambda i: (i, 0)),
    out_shape=jax.ShapeDtypeStruct((NPAD, D), jnp.float32),
)

_final = pl.pallas_call(
    _final_body,
    grid=(_GRID,),
    in_specs=[
        pl.BlockSpec((NC, _BLK, D), lambda i: (0, i, 0)),
        pl.BlockSpec((_BLK,), lambda i: (i,)),
        pl.BlockSpec((_BLK, D), lambda i: (i, 0)),
        pl.BlockSpec((D, D), lambda i: (0, 0)),
        pl.BlockSpec((D, D), lambda i: (0, 0)),
        pl.BlockSpec((D,), lambda i: (0,)),
        pl.BlockSpec((D,), lambda i: (0,)),
    ],
    out_specs=pl.BlockSpec((_BLK, D), lambda i: (i, 0)),
    out_shape=jax.ShapeDtypeStruct((NPAD, D), jnp.float32),
)


@jax.jit
def kernel(feat, edge_index, W_fc, b_fc, W_res, b_res):
    src = jnp.concatenate(
        [edge_index[0], jnp.zeros((E_PAD - E,), jnp.int32)])
    pad_dst = N + (jnp.arange(E_PAD - E, dtype=jnp.int32) % (NPAD - N))
    dst = jnp.concatenate([edge_index[1], pad_dst])
    src4 = src.reshape(NC, NS, CHUNKS_PER_W, CHUNK)
    dst4 = dst.reshape(NC, NS, CHUNKS_PER_W, CHUNK)
    dst3 = dst.reshape(NC, NS, EW)

    feat_pad = jnp.zeros((NPAD, D), jnp.float32).at[:N].set(feat)
    zeros_blk = jnp.zeros((ROWS_PER_TILE, D), jnp.float32)

    deg_parts = _deg_kernel(dst3)
    x0, norm, norm2 = _norm_x0(deg_parts, feat_pad)

    s1 = _hop_kernel(x0, src4, dst4, zeros_blk)
    x1 = _scale_sum(s1, norm2)
    s2 = _hop_kernel(x1, src4, dst4, zeros_blk)

    out = _final(s2, norm, feat_pad, W_fc, W_res, b_fc, b_res)
    return out[:N]
